# baseline (speedup 1.0000x reference)
import functools

import jax
import jax.numpy as jnp
from jax import lax
from jax.experimental import pallas as pl
from jax.experimental.pallas import tpu as pltpu
from jax.experimental.pallas import tpu_sc as plsc

_NBUF = 3


def _fp_kernel(mask_ref, out_ref):
    n = mask_ref.shape[1]
    iota = jax.lax.broadcasted_iota(jnp.int32, mask_ref.shape, 1)
    cand = jnp.where(mask_ref[...] != 0, n, iota)
    fp = jnp.min(cand, axis=1, keepdims=True)
    out_ref[...] = jnp.broadcast_to(fp, out_ref.shape)


def _sc_body(b, n, d, x_hbm, out_hbm, g0, g1, g2, o0, o1, o2, zbuf, cbuf,
             sem_in, sem_out, sem_z):
    gbufs = [g0, g1, g2]
    obufs = [o0, o1, o2]
    c = lax.axis_index("c")
    s = lax.axis_index("s")
    bpc = b // 2
    npc = 16 // bpc
    batch = c * bpc + s // npc
    sl = s % npc
    ntr = n // 8
    tpw = ntr // npc
    t0 = sl * tpw
    is_last = sl == npc - 1

    def gather(tr, k):
        trc = jnp.maximum(tr, 0)
        pltpu.make_async_copy(
            x_hbm.at[batch, pl.ds(trc * 8, 8), :], gbufs[k], sem_in).start()

    def wait_gather(k):
        pltpu.make_async_copy(
            x_hbm.at[batch, pl.ds(0, 8), :], gbufs[k], sem_in).wait()

    def wait_scatter(k):
        pltpu.make_async_copy(
            obufs[k], out_hbm.at[batch, pl.ds(0, 8), :], sem_out).wait()

    gather(t0 - 1, _NBUF - 1)
    wait_gather(_NBUF - 1)

    def prime_cc(cc, carry):
        cbuf[0, pl.ds(cc * 16, 16)] = gbufs[_NBUF - 1][7, pl.ds(cc * 16, 16)]
        return carry

    lax.fori_loop(0, d // 16, prime_cc, 0)

    groups = tpw // _NBUF

    def copy_group(g, carry):
        base = g * _NBUF
        for k in range(_NBUF):
            pl.when(g > 0)(lambda k=k: wait_scatter(k))
            gather(t0 + base + k, k)
        for k in range(_NBUF):
            wait_gather(k)

            def shuf_cc(cc, carry, k=k):
                col = pl.ds(cc * 16, 16)
                v7 = gbufs[k][7, col]
                obufs[k][0, col] = cbuf[0, col]
                for r in range(7):
                    obufs[k][r + 1, col] = gbufs[k][r, col]
                cbuf[0, col] = v7
                return carry

            lax.fori_loop(0, d // 16, shuf_cc, 0)
            pltpu.make_async_copy(
                obufs[k],
                out_hbm.at[batch, pl.ds((t0 + base + k) * 8, 8), :],
                sem_out).start()
        return carry

    lax.fori_loop(0, groups, copy_group, 0)
    for k in range(_NBUF):
        wait_scatter(k)

    @pl.when(is_last)
    def _():
        zv = jnp.zeros((16,), jnp.float32)

        def z_cc(cc, carry):
            zbuf[0, pl.ds(cc * 16, 16)] = cbuf[0, pl.ds(cc * 16, 16)]
            zbuf[1, pl.ds(cc * 16, 16)] = zv
            return carry

        lax.fori_loop(0, d // 16, z_cc, 0)
        cp = pltpu.make_async_copy(
            zbuf, out_hbm.at[batch, pl.ds(n, 2), :], sem_z)
        cp.start()
        cp.wait()


def _fixup_kernel(n, fp_ref, in_ref, start_ref, end_ref, out_ref):
    bi = pl.program_id(0)
    m = pl.program_id(1)
    tr = jnp.where(m == 0, 0, jnp.where(m == 1, fp_ref[bi] // 8, (n + 1) // 8))
    rows = jax.lax.broadcasted_iota(jnp.int32, (8, 1), 0) + tr * 8
    v = in_ref[...]
    v = jnp.where(rows == 0, start_ref[...], v)
    v = jnp.where(rows == n + 1, 0.0, v)
    v = jnp.where(rows == fp_ref[bi], end_ref[...], v)
    out_ref[...] = v


def kernel(x, protein_mask, start, end):
    b, n, d = x.shape
    mask_i32 = protein_mask.astype(jnp.int32)
    fp_full = pl.pallas_call(
        _fp_kernel,
        out_shape=jax.ShapeDtypeStruct((b, 128), jnp.int32),
    )(mask_i32)
    fp = fp_full[:, 0]

    sc_call = pl.kernel(
        functools.partial(_sc_body, b, n, d),
        out_type=jax.ShapeDtypeStruct((b, n + 2, d), jnp.float32),
        mesh=plsc.VectorSubcoreMesh(core_axis_name="c", subcore_axis_name="s"),
        scratch_types=[
            pltpu.VMEM((8, d), jnp.float32),
            pltpu.VMEM((8, d), jnp.float32),
            pltpu.VMEM((8, d), jnp.float32),
            pltpu.VMEM((8, d), jnp.float32),
            pltpu.VMEM((8, d), jnp.float32),
            pltpu.VMEM((8, d), jnp.float32),
            pltpu.VMEM((2, d), jnp.float32),
            pltpu.VMEM((1, d), jnp.float32),
            pltpu.SemaphoreType.DMA,
            pltpu.SemaphoreType.DMA,
            pltpu.SemaphoreType.DMA,
        ],
    )
    out0 = sc_call(x)

    def tr_map(bi, m, fp_ref):
        return (bi, jnp.where(m == 0, 0,
                              jnp.where(m == 1, fp_ref[bi] // 8,
                                        (n + 1) // 8)), 0)

    out = pl.pallas_call(
        functools.partial(_fixup_kernel, n),
        grid_spec=pltpu.PrefetchScalarGridSpec(
            num_scalar_prefetch=1,
            grid=(b, 3),
            in_specs=[
                pl.BlockSpec((None, 8, d), tr_map),
                pl.BlockSpec((1, d), lambda bi, m, *_: (0, 0)),
                pl.BlockSpec((1, d), lambda bi, m, *_: (0, 0)),
            ],
            out_specs=pl.BlockSpec((None, 8, d), tr_map),
        ),
        out_shape=jax.ShapeDtypeStruct((b, n + 2, d), jnp.float32),
        input_output_aliases={1: 0},
    )(fp, out0, start.reshape(1, d), end.reshape(1, d))
    return out
